# Initial kernel scaffold; baseline (speedup 1.0000x reference)
import jax
import jax.numpy as jnp
from jax.experimental import pallas as pl


def kernel(input_ids, polarity_ids, intensity_ids, word_table, polarity_table, intensity_table, W, b, gamma, beta):
    raise NotImplementedError("write your pallas kernel here")


# speedup vs baseline: 1.7733x; 1.7733x over previous
import functools

import jax
import jax.numpy as jnp
from jax import lax
from jax.experimental import pallas as pl
from jax.experimental.pallas import tpu as pltpu
from jax.experimental.pallas import tpu_sc as plsc

_GATHER_WINDOW = 128
_TILE = 2048


def _sc_gather(word_table, idx_flat):
    n = idx_flat.shape[0]
    d = word_table.shape[1]
    idx2 = idx_flat.reshape(1, n)
    mesh = plsc.VectorSubcoreMesh(core_axis_name="c", subcore_axis_name="s")

    @functools.partial(
        pl.kernel,
        out_type=jax.ShapeDtypeStruct((n, d), word_table.dtype),
        mesh=mesh,
    )
    def gather_kernel(tab_hbm, i_hbm, o_hbm):
        def body(i_vmem, o_vmem):
            pltpu.sync_copy(tab_hbm.at[i_vmem.at[0]], o_vmem)

        pltpu.emit_pipeline(
            body,
            grid=(n // _GATHER_WINDOW,),
            in_specs=[pl.BlockSpec((1, _GATHER_WINDOW), index_map=lambda i: (0, i))],
            out_specs=[pl.BlockSpec((_GATHER_WINDOW, d), index_map=lambda i: (i, 0))],
            core_axis_name=("c", "s"),
            dimension_semantics=(pltpu.PARALLEL,),
        )(i_hbm, o_hbm)

    return gather_kernel(word_table, idx2)


def _fused_body(g_ref, c_ref, w_ref, b_ref, pp_ref, ip_ref, gm_ref, bt_ref, o_ref):
    d = g_ref.shape[1] // 2
    tile = g_ref.shape[0]
    hi = lax.Precision.HIGHEST
    w_w = w_ref[0:d, :]
    w_p = w_ref[d : 2 * d, :]
    w_i = w_ref[2 * d : 3 * d, :]
    pw = jnp.dot(pp_ref[...], w_p, preferred_element_type=jnp.float32, precision=hi)
    iw = jnp.dot(ip_ref[...], w_i, preferred_element_type=jnp.float32, precision=hi)
    row = lax.broadcasted_iota(jnp.int32, (16, 8), 0)
    col = lax.broadcasted_iota(jnp.int32, (16, 8), 1)
    sel_p = (col == row // 5).astype(jnp.float32)
    sel_i = (col == row % 5).astype(jnp.float32)
    combo = jnp.dot(sel_p, pw, preferred_element_type=jnp.float32, precision=hi)
    combo = combo + jnp.dot(sel_i, iw, preferred_element_type=jnp.float32, precision=hi)
    meta = c_ref[0]
    cid = jnp.bitwise_and(meta, 15)
    par = lax.shift_right_logical(meta, 4)
    oh = (cid == lax.broadcasted_iota(jnp.int32, (tile, 16), 1)).astype(jnp.float32)
    g64 = jnp.where(par > 0, g_ref[:, d:2 * d], g_ref[:, 0:d])
    h = jnp.dot(g64, w_w, preferred_element_type=jnp.float32, precision=hi)
    h = h + jnp.dot(oh, combo, preferred_element_type=jnp.float32, precision=hi)
    h = h + b_ref[...]
    h = 0.5 * h * (1.0 + lax.erf(h * 0.7071067811865476))
    mean = jnp.mean(h, axis=-1, keepdims=True)
    cent = h - mean
    var = jnp.mean(cent * cent, axis=-1, keepdims=True)
    o_ref[...] = cent * lax.rsqrt(var + 1e-5) * gm_ref[...] + bt_ref[...]


def _tc_fused(gathered, meta, W, b2, pol_pad, int_pad, gamma2, beta2):
    n, d2 = gathered.shape
    d = d2 // 2
    grid = n // _TILE
    meta3 = meta.reshape(grid, _TILE, 1)
    return pl.pallas_call(
        _fused_body,
        grid=(grid,),
        in_specs=[
            pl.BlockSpec((_TILE, d2), lambda i: (i, 0)),
            pl.BlockSpec((1, _TILE, 1), lambda i: (i, 0, 0)),
            pl.BlockSpec((3 * d, d), lambda i: (0, 0)),
            pl.BlockSpec((1, d), lambda i: (0, 0)),
            pl.BlockSpec((8, d), lambda i: (0, 0)),
            pl.BlockSpec((8, d), lambda i: (0, 0)),
            pl.BlockSpec((1, d), lambda i: (0, 0)),
            pl.BlockSpec((1, d), lambda i: (0, 0)),
        ],
        out_specs=pl.BlockSpec((_TILE, d), lambda i: (i, 0)),
        out_shape=jax.ShapeDtypeStruct((n, d), jnp.float32),
    )(gathered, meta3, W, b2, pol_pad, int_pad, gamma2, beta2)


def kernel(input_ids, polarity_ids, intensity_ids, word_table, polarity_table,
           intensity_table, W, b, gamma, beta):
    bsz, seq = input_ids.shape
    d = word_table.shape[1]
    idx_flat = input_ids.reshape(-1).astype(jnp.int32)
    cid = (polarity_ids.astype(jnp.int32) * 5
           + intensity_ids.astype(jnp.int32)).reshape(-1)
    meta = jnp.bitwise_and(idx_flat, 1) * 16 + cid
    pol_pad = jnp.zeros((8, d), jnp.float32).at[:polarity_table.shape[0]].set(
        polarity_table)
    int_pad = jnp.zeros((8, d), jnp.float32).at[:intensity_table.shape[0]].set(
        intensity_table)
    table2 = word_table.reshape(-1, 2 * d)
    gathered = _sc_gather(table2, lax.shift_right_logical(idx_flat, 1))
    out = _tc_fused(gathered, meta, W, b.reshape(1, d), pol_pad, int_pad,
                    gamma.reshape(1, d), beta.reshape(1, d))
    return out.reshape(bsz, seq, d)


# speedup vs baseline: 2.1093x; 1.1894x over previous
import functools

import jax
import jax.numpy as jnp
from jax import lax
from jax.experimental import pallas as pl
from jax.experimental.pallas import tpu as pltpu
from jax.experimental.pallas import tpu_sc as plsc

_GATHER_WINDOW = 128
_TILE = 3200
_NBUF = 5


def _sc_gather(word_table, idx_flat):
    n = idx_flat.shape[0]
    d = word_table.shape[1]
    mesh = plsc.VectorSubcoreMesh(core_axis_name="c", subcore_axis_name="s")
    nunits = mesh.num_cores * mesh.num_subcores
    per_unit = n // nunits
    nwin = per_unit // _GATHER_WINDOW
    assert per_unit % _GATHER_WINDOW == 0 and nwin % _NBUF == 0

    @functools.partial(
        pl.kernel,
        out_type=jax.ShapeDtypeStruct((n, d), word_table.dtype),
        mesh=mesh,
        scratch_types=[
            pltpu.VMEM((per_unit,), jnp.int32),
            pltpu.VMEM((_NBUF, _GATHER_WINDOW, d), jnp.float32),
            pltpu.SemaphoreType.DMA,
            pltpu.SemaphoreType.DMA((_NBUF,)),
        ],
    )
    def gather_kernel(tab_hbm, i_hbm, o_hbm, idx_v, rows_v, isem, gsem):
        wid = jax.lax.axis_index("s") * mesh.num_cores + jax.lax.axis_index("c")
        base = wid * per_unit
        pltpu.async_copy(i_hbm.at[pl.ds(base, per_unit)], idx_v, isem).wait()
        for bb in range(_NBUF):
            pltpu.async_copy(
                tab_hbm.at[idx_v.at[pl.ds(bb * _GATHER_WINDOW, _GATHER_WINDOW)]],
                rows_v.at[bb], gsem.at[bb])

        @pl.loop(0, nwin, step=_NBUF)
        def _(w0):
            for bb in range(_NBUF):
                w = w0 + bb
                pltpu.make_async_copy(
                    tab_hbm.at[idx_v.at[pl.ds(0, _GATHER_WINDOW)]],
                    rows_v.at[bb], gsem.at[bb]).wait()
                pltpu.sync_copy(
                    rows_v.at[bb],
                    o_hbm.at[pl.ds(base + w * _GATHER_WINDOW, _GATHER_WINDOW)])

                @pl.when(w + _NBUF < nwin)
                def _():
                    pltpu.async_copy(
                        tab_hbm.at[idx_v.at[pl.ds((w + _NBUF) * _GATHER_WINDOW,
                                                  _GATHER_WINDOW)]],
                        rows_v.at[bb], gsem.at[bb])

    return gather_kernel(word_table, idx_flat)


def _fused_body(g_ref, oh_ref, w_ref, b_ref, pp_ref, ip_ref, gm_ref, bt_ref,
                o_ref):
    d = g_ref.shape[1] // 2
    w_w = w_ref[0:d, :]
    w_p = w_ref[d:2 * d, :]
    w_i = w_ref[2 * d:3 * d, :]
    pw = jnp.dot(pp_ref[...], w_p, preferred_element_type=jnp.float32)
    iw = jnp.dot(ip_ref[...], w_i, preferred_element_type=jnp.float32)
    row = lax.broadcasted_iota(jnp.int32, (32, 8), 0)
    col = lax.broadcasted_iota(jnp.int32, (32, 8), 1)
    cid16 = jnp.bitwise_and(row, 15)
    sel_p = (col == cid16 // 5).astype(jnp.float32)
    sel_i = (col == cid16 % 5).astype(jnp.float32)
    c32 = (jnp.dot(sel_p, pw, preferred_element_type=jnp.float32)
           + jnp.dot(sel_i, iw, preferred_element_type=jnp.float32)
           + b_ref[...])
    ppar = (lax.broadcasted_iota(jnp.int32, (32, d), 0) >= 16).astype(
        jnp.float32)
    oh = oh_ref[...].astype(jnp.float32)
    par64 = jnp.dot(oh, ppar, preferred_element_type=jnp.float32)
    gl = g_ref[:, 0:d]
    gr = g_ref[:, d:2 * d]
    sel = gl + par64 * (gr - gl)
    h = jnp.dot(sel, w_w, preferred_element_type=jnp.float32)
    h = h + jnp.dot(oh, c32, preferred_element_type=jnp.float32)
    h = 0.5 * h * (1.0 + lax.erf(h * 0.7071067811865476))
    mean = jnp.mean(h, axis=-1, keepdims=True)
    cent = h - mean
    var = jnp.mean(cent * cent, axis=-1, keepdims=True)
    res = cent * lax.rsqrt(var + 1e-5) * gm_ref[...] + bt_ref[...]
    o_ref[...] = res.reshape(o_ref.shape)


def _tc_fused(gathered, oh, W, b2, pol_pad, int_pad, gamma2, beta2, bsz, seq):
    n, d2 = gathered.shape
    d = d2 // 2
    grid = n // _TILE
    brows = _TILE // seq
    return pl.pallas_call(
        _fused_body,
        grid=(grid,),
        in_specs=[
            pl.BlockSpec((_TILE, d2), lambda i: (i, 0)),
            pl.BlockSpec((_TILE, 32), lambda i: (i, 0)),
            pl.BlockSpec((3 * d, d), lambda i: (0, 0)),
            pl.BlockSpec((1, d), lambda i: (0, 0)),
            pl.BlockSpec((8, d), lambda i: (0, 0)),
            pl.BlockSpec((8, d), lambda i: (0, 0)),
            pl.BlockSpec((1, d), lambda i: (0, 0)),
            pl.BlockSpec((1, d), lambda i: (0, 0)),
        ],
        out_specs=pl.BlockSpec((brows, seq, d), lambda i: (i, 0, 0)),
        out_shape=jax.ShapeDtypeStruct((bsz, seq, d), jnp.float32),
    )(gathered, oh, W, b2, pol_pad, int_pad, gamma2, beta2)


def kernel(input_ids, polarity_ids, intensity_ids, word_table, polarity_table,
           intensity_table, W, b, gamma, beta):
    bsz, seq = input_ids.shape
    d = word_table.shape[1]
    idx_flat = input_ids.reshape(-1).astype(jnp.int32)
    cid = (polarity_ids.astype(jnp.int32) * 5
           + intensity_ids.astype(jnp.int32)).reshape(-1)
    ecid = jnp.bitwise_and(idx_flat, 1) * 16 + cid
    oh = jax.nn.one_hot(ecid, 32, dtype=jnp.bfloat16)
    pol_pad = jnp.zeros((8, d), jnp.float32).at[:polarity_table.shape[0]].set(
        polarity_table)
    int_pad = jnp.zeros((8, d), jnp.float32).at[:intensity_table.shape[0]].set(
        intensity_table)
    table2 = word_table.reshape(-1, 2 * d)
    gathered = _sc_gather(table2, lax.shift_right_logical(idx_flat, 1))
    return _tc_fused(gathered, oh, W, b.reshape(1, d), pol_pad, int_pad,
                     gamma.reshape(1, d), beta.reshape(1, d), bsz, seq)


# speedup vs baseline: 2.1417x; 1.0154x over previous
import functools

import jax
import jax.numpy as jnp
from jax import lax
from jax.experimental import pallas as pl
from jax.experimental.pallas import tpu as pltpu
from jax.experimental.pallas import tpu_sc as plsc

_GATHER_WINDOW = 128
_TILE = 3200
_NBUF = 5
_RELAYOUT_ROWS = 5000


def _relayout_body(a_ref, b_ref, o_ref):
    d = a_ref.shape[1]
    o_ref[:, 0:d] = a_ref[...]
    o_ref[:, d:2 * d] = b_ref[...]


def _tc_relayout(tab):
    v, d = tab.shape
    half_blocks = (v // 2) // _RELAYOUT_ROWS
    grid = half_blocks
    return pl.pallas_call(
        _relayout_body,
        grid=(grid,),
        in_specs=[
            pl.BlockSpec((_RELAYOUT_ROWS, d), lambda i: (i, 0)),
            pl.BlockSpec((_RELAYOUT_ROWS, d),
                         lambda i: (i + half_blocks, 0)),
        ],
        out_specs=pl.BlockSpec((_RELAYOUT_ROWS, 2 * d), lambda i: (i, 0)),
        out_shape=jax.ShapeDtypeStruct((v // 2, 2 * d), jnp.float32),
        compiler_params=pltpu.CompilerParams(
            dimension_semantics=("parallel",)),
    )(tab, tab)


def _sc_gather(word_table, idx_flat):
    n = idx_flat.shape[0]
    d = word_table.shape[1]
    mesh = plsc.VectorSubcoreMesh(core_axis_name="c", subcore_axis_name="s")
    nunits = mesh.num_cores * mesh.num_subcores
    per_unit = n // nunits
    nwin = per_unit // _GATHER_WINDOW
    assert per_unit % _GATHER_WINDOW == 0 and nwin % _NBUF == 0

    @functools.partial(
        pl.kernel,
        out_type=jax.ShapeDtypeStruct((n, d), word_table.dtype),
        mesh=mesh,
        scratch_types=[
            pltpu.VMEM((per_unit,), jnp.int32),
            pltpu.VMEM((_NBUF, _GATHER_WINDOW, d), jnp.float32),
            pltpu.SemaphoreType.DMA,
            pltpu.SemaphoreType.DMA((_NBUF,)),
        ],
    )
    def gather_kernel(tab_hbm, i_hbm, o_hbm, idx_v, rows_v, isem, gsem):
        wid = jax.lax.axis_index("s") * mesh.num_cores + jax.lax.axis_index("c")
        base = wid * per_unit
        pltpu.async_copy(i_hbm.at[pl.ds(base, per_unit)], idx_v, isem).wait()
        for bb in range(_NBUF):
            pltpu.async_copy(
                tab_hbm.at[idx_v.at[pl.ds(bb * _GATHER_WINDOW, _GATHER_WINDOW)]],
                rows_v.at[bb], gsem.at[bb])

        @pl.loop(0, nwin, step=_NBUF)
        def _(w0):
            for bb in range(_NBUF):
                w = w0 + bb
                pltpu.make_async_copy(
                    tab_hbm.at[idx_v.at[pl.ds(0, _GATHER_WINDOW)]],
                    rows_v.at[bb], gsem.at[bb]).wait()
                pltpu.sync_copy(
                    rows_v.at[bb],
                    o_hbm.at[pl.ds(base + w * _GATHER_WINDOW, _GATHER_WINDOW)])

                @pl.when(w + _NBUF < nwin)
                def _():
                    pltpu.async_copy(
                        tab_hbm.at[idx_v.at[pl.ds((w + _NBUF) * _GATHER_WINDOW,
                                                  _GATHER_WINDOW)]],
                        rows_v.at[bb], gsem.at[bb])

    return gather_kernel(word_table, idx_flat)


def _fused_body(g_ref, oh_ref, w_ref, b_ref, pp_ref, ip_ref, gm_ref, bt_ref,
                o_ref):
    d = g_ref.shape[1] // 2
    w_w = w_ref[0:d, :]
    w_p = w_ref[d:2 * d, :]
    w_i = w_ref[2 * d:3 * d, :]
    pw = jnp.dot(pp_ref[...], w_p, preferred_element_type=jnp.float32)
    iw = jnp.dot(ip_ref[...], w_i, preferred_element_type=jnp.float32)
    row = lax.broadcasted_iota(jnp.int32, (32, 8), 0)
    col = lax.broadcasted_iota(jnp.int32, (32, 8), 1)
    cid16 = jnp.bitwise_and(row, 15)
    sel_p = (col == cid16 // 5).astype(jnp.float32)
    sel_i = (col == cid16 % 5).astype(jnp.float32)
    c32 = (jnp.dot(sel_p, pw, preferred_element_type=jnp.float32)
           + jnp.dot(sel_i, iw, preferred_element_type=jnp.float32)
           + b_ref[...])
    ppar = (lax.broadcasted_iota(jnp.int32, (32, d), 0) >= 16).astype(
        jnp.float32)
    oh = oh_ref[...].astype(jnp.float32)
    par64 = jnp.dot(oh, ppar, preferred_element_type=jnp.float32)
    gl = g_ref[:, 0:d]
    gr = g_ref[:, d:2 * d]
    sel = gl + par64 * (gr - gl)
    h = jnp.dot(sel, w_w, preferred_element_type=jnp.float32)
    h = h + jnp.dot(oh, c32, preferred_element_type=jnp.float32)
    h = 0.5 * h * (1.0 + lax.erf(h * 0.7071067811865476))
    mean = jnp.mean(h, axis=-1, keepdims=True)
    cent = h - mean
    var = jnp.mean(cent * cent, axis=-1, keepdims=True)
    res = cent * lax.rsqrt(var + 1e-5) * gm_ref[...] + bt_ref[...]
    o_ref[...] = res.reshape(o_ref.shape)


def _tc_fused(gathered, oh, W, b2, pol_pad, int_pad, gamma2, beta2, bsz, seq):
    n, d2 = gathered.shape
    d = d2 // 2
    grid = n // _TILE
    brows = _TILE // seq
    return pl.pallas_call(
        _fused_body,
        grid=(grid,),
        in_specs=[
            pl.BlockSpec((_TILE, d2), lambda i: (i, 0)),
            pl.BlockSpec((_TILE, 32), lambda i: (i, 0)),
            pl.BlockSpec((3 * d, d), lambda i: (0, 0)),
            pl.BlockSpec((1, d), lambda i: (0, 0)),
            pl.BlockSpec((8, d), lambda i: (0, 0)),
            pl.BlockSpec((8, d), lambda i: (0, 0)),
            pl.BlockSpec((1, d), lambda i: (0, 0)),
            pl.BlockSpec((1, d), lambda i: (0, 0)),
        ],
        out_specs=pl.BlockSpec((brows, seq, d), lambda i: (i, 0, 0)),
        out_shape=jax.ShapeDtypeStruct((bsz, seq, d), jnp.float32),
        compiler_params=pltpu.CompilerParams(
            dimension_semantics=("parallel",)),
    )(gathered, oh, W, b2, pol_pad, int_pad, gamma2, beta2)


def kernel(input_ids, polarity_ids, intensity_ids, word_table, polarity_table,
           intensity_table, W, b, gamma, beta):
    bsz, seq = input_ids.shape
    d = word_table.shape[1]
    v = word_table.shape[0]
    idx_flat = input_ids.reshape(-1).astype(jnp.int32)
    par = (idx_flat >= v // 2).astype(jnp.int32)
    pair_idx = idx_flat - par * (v // 2)
    cid = (polarity_ids.astype(jnp.int32) * 5
           + intensity_ids.astype(jnp.int32)).reshape(-1)
    ecid = par * 16 + cid
    oh = jax.nn.one_hot(ecid, 32, dtype=jnp.bfloat16)
    pol_pad = jnp.zeros((8, d), jnp.float32).at[:polarity_table.shape[0]].set(
        polarity_table)
    int_pad = jnp.zeros((8, d), jnp.float32).at[:intensity_table.shape[0]].set(
        intensity_table)
    table2 = _tc_relayout(word_table)
    gathered = _sc_gather(table2, pair_idx)
    return _tc_fused(gathered, oh, W, b.reshape(1, d), pol_pad, int_pad,
                     gamma.reshape(1, d), beta.reshape(1, d), bsz, seq)


# speedup vs baseline: 2.4905x; 1.1628x over previous
import functools

import jax
import jax.numpy as jnp
from jax import lax
from jax.experimental import pallas as pl
from jax.experimental.pallas import tpu as pltpu
from jax.experimental.pallas import tpu_sc as plsc

_GATHER_WINDOW = 128
_TILE = 3200
_NBUF = 5
_RELAYOUT_ROWS = 5000


def _relayout_body(t_ref, o_ref):
    d = t_ref.shape[2]
    o_ref[:, 0:d] = t_ref[0]
    o_ref[:, d:2 * d] = t_ref[1]


def _tc_relayout(tab):
    v, d = tab.shape
    tab3 = tab.reshape(2, v // 2, d)
    grid = (v // 2) // _RELAYOUT_ROWS
    return pl.pallas_call(
        _relayout_body,
        grid=(grid,),
        in_specs=[pl.BlockSpec((2, _RELAYOUT_ROWS, d), lambda i: (0, i, 0))],
        out_specs=pl.BlockSpec((_RELAYOUT_ROWS, 2 * d), lambda i: (i, 0)),
        out_shape=jax.ShapeDtypeStruct((v // 2, 2 * d), jnp.float32),
        compiler_params=pltpu.CompilerParams(
            dimension_semantics=("parallel",)),
    )(tab3)


def _sc_gather(word_table, idx_flat):
    n = idx_flat.shape[0]
    d = word_table.shape[1]
    mesh = plsc.VectorSubcoreMesh(core_axis_name="c", subcore_axis_name="s")
    nunits = mesh.num_cores * mesh.num_subcores
    per_unit = n // nunits
    nwin = per_unit // _GATHER_WINDOW
    assert per_unit % _GATHER_WINDOW == 0 and nwin % _NBUF == 0

    @functools.partial(
        pl.kernel,
        out_type=jax.ShapeDtypeStruct((n, d), word_table.dtype),
        mesh=mesh,
        scratch_types=[
            pltpu.VMEM((per_unit,), jnp.int32),
            pltpu.VMEM((_NBUF, _GATHER_WINDOW, d), jnp.float32),
            pltpu.SemaphoreType.DMA,
            pltpu.SemaphoreType.DMA((_NBUF,)),
        ],
    )
    def gather_kernel(tab_hbm, i_hbm, o_hbm, idx_v, rows_v, isem, gsem):
        wid = jax.lax.axis_index("s") * mesh.num_cores + jax.lax.axis_index("c")
        base = wid * per_unit
        pltpu.async_copy(i_hbm.at[pl.ds(base, per_unit)], idx_v, isem).wait()
        for bb in range(_NBUF):
            pltpu.async_copy(
                tab_hbm.at[idx_v.at[pl.ds(bb * _GATHER_WINDOW, _GATHER_WINDOW)]],
                rows_v.at[bb], gsem.at[bb])

        @pl.loop(0, nwin, step=_NBUF)
        def _(w0):
            for bb in range(_NBUF):
                w = w0 + bb
                pltpu.make_async_copy(
                    tab_hbm.at[idx_v.at[pl.ds(0, _GATHER_WINDOW)]],
                    rows_v.at[bb], gsem.at[bb]).wait()
                pltpu.sync_copy(
                    rows_v.at[bb],
                    o_hbm.at[pl.ds(base + w * _GATHER_WINDOW, _GATHER_WINDOW)])

                @pl.when(w + _NBUF < nwin)
                def _():
                    pltpu.async_copy(
                        tab_hbm.at[idx_v.at[pl.ds((w + _NBUF) * _GATHER_WINDOW,
                                                  _GATHER_WINDOW)]],
                        rows_v.at[bb], gsem.at[bb])

    return gather_kernel(word_table, idx_flat)


def _fused_body(g_ref, oh_ref, w_ref, b_ref, pp_ref, ip_ref, gm_ref, bt_ref,
                o_ref):
    d = g_ref.shape[1] // 2
    w_w = w_ref[0:d, :]
    w_p = w_ref[d:2 * d, :]
    w_i = w_ref[2 * d:3 * d, :]
    pw = jnp.dot(pp_ref[...], w_p, preferred_element_type=jnp.float32)
    iw = jnp.dot(ip_ref[...], w_i, preferred_element_type=jnp.float32)
    row = lax.broadcasted_iota(jnp.int32, (32, 8), 0)
    col = lax.broadcasted_iota(jnp.int32, (32, 8), 1)
    cid16 = jnp.bitwise_and(row, 15)
    sel_p = (col == cid16 // 5).astype(jnp.float32)
    sel_i = (col == cid16 % 5).astype(jnp.float32)
    c32 = (jnp.dot(sel_p, pw, preferred_element_type=jnp.float32)
           + jnp.dot(sel_i, iw, preferred_element_type=jnp.float32)
           + b_ref[...])
    ppar = (lax.broadcasted_iota(jnp.int32, (32, d), 0) >= 16).astype(
        jnp.float32)
    oh = oh_ref[...].astype(jnp.float32)
    par64 = jnp.dot(oh, ppar, preferred_element_type=jnp.float32)
    gl = g_ref[:, 0:d]
    gr = g_ref[:, d:2 * d]
    sel = gl + par64 * (gr - gl)
    h = jnp.dot(sel, w_w, preferred_element_type=jnp.float32)
    h = h + jnp.dot(oh, c32, preferred_element_type=jnp.float32)
    h = 0.5 * h * (1.0 + lax.erf(h * 0.7071067811865476))
    mean = jnp.mean(h, axis=-1, keepdims=True)
    cent = h - mean
    var = jnp.mean(cent * cent, axis=-1, keepdims=True)
    res = cent * lax.rsqrt(var + 1e-5) * gm_ref[...] + bt_ref[...]
    o_ref[...] = res.reshape(o_ref.shape)


def _tc_fused(gathered, oh, W, b2, pol_pad, int_pad, gamma2, beta2, bsz, seq):
    n, d2 = gathered.shape
    d = d2 // 2
    grid = n // _TILE
    brows = _TILE // seq
    return pl.pallas_call(
        _fused_body,
        grid=(grid,),
        in_specs=[
            pl.BlockSpec((_TILE, d2), lambda i: (i, 0)),
            pl.BlockSpec((_TILE, 32), lambda i: (i, 0)),
            pl.BlockSpec((3 * d, d), lambda i: (0, 0)),
            pl.BlockSpec((1, d), lambda i: (0, 0)),
            pl.BlockSpec((8, d), lambda i: (0, 0)),
            pl.BlockSpec((8, d), lambda i: (0, 0)),
            pl.BlockSpec((1, d), lambda i: (0, 0)),
            pl.BlockSpec((1, d), lambda i: (0, 0)),
        ],
        out_specs=pl.BlockSpec((brows, seq, d), lambda i: (i, 0, 0)),
        out_shape=jax.ShapeDtypeStruct((bsz, seq, d), jnp.float32),
        compiler_params=pltpu.CompilerParams(
            dimension_semantics=("parallel",)),
    )(gathered, oh, W, b2, pol_pad, int_pad, gamma2, beta2)


def kernel(input_ids, polarity_ids, intensity_ids, word_table, polarity_table,
           intensity_table, W, b, gamma, beta):
    bsz, seq = input_ids.shape
    d = word_table.shape[1]
    v = word_table.shape[0]
    idx_flat = input_ids.reshape(-1).astype(jnp.int32)
    par = (idx_flat >= v // 2).astype(jnp.int32)
    pair_idx = idx_flat - par * (v // 2)
    cid = (polarity_ids.astype(jnp.int32) * 5
           + intensity_ids.astype(jnp.int32)).reshape(-1)
    ecid = par * 16 + cid
    oh = jax.nn.one_hot(ecid, 32, dtype=jnp.bfloat16)
    pol_pad = jnp.zeros((8, d), jnp.float32).at[:polarity_table.shape[0]].set(
        polarity_table)
    int_pad = jnp.zeros((8, d), jnp.float32).at[:intensity_table.shape[0]].set(
        intensity_table)
    table2 = _tc_relayout(word_table)
    gathered = _sc_gather(table2, pair_idx)
    return _tc_fused(gathered, oh, W, b.reshape(1, d), pol_pad, int_pad,
                     gamma.reshape(1, d), beta.reshape(1, d), bsz, seq)
